# Initial kernel scaffold; baseline (speedup 1.0000x reference)
import jax
import jax.numpy as jnp
from jax.experimental import pallas as pl


def kernel(feature, coord, params):
    raise NotImplementedError("write your pallas kernel here")


# speedup vs baseline: 6.7397x; 6.7397x over previous
import functools

import jax
import jax.numpy as jnp
from jax.experimental import pallas as pl

_NG = 1024
_KS = (16, 32, 64)
_RS = (0.1, 0.2, 0.4)
_ROWS = 512

_F32 = jnp.float32


def _fps_kernel(coord_ref, ce_ref):
    b, _, n = coord_ref.shape
    s = ce_ref.shape[1]
    x0 = coord_ref[:, 0, :]
    x1 = coord_ref[:, 1, :]
    x2 = coord_ref[:, 2, :]
    iota = jax.lax.broadcasted_iota(jnp.int32, (b, n), 1)

    def body(i, carry):
        dist, far = carry
        oh = (iota == far).astype(_F32)
        c0 = jnp.sum(x0 * oh, axis=1, keepdims=True)
        c1 = jnp.sum(x1 * oh, axis=1, keepdims=True)
        c2 = jnp.sum(x2 * oh, axis=1, keepdims=True)
        cent = jnp.concatenate([c0, c1, c2], axis=1)
        ce_ref[:, pl.ds(i, 1), :] = cent[:, None, :]
        d0 = x0 - c0
        d1 = x1 - c1
        d2 = x2 - c2
        d = (d0 * d0 + d1 * d1) + d2 * d2
        dist = jnp.minimum(dist, d)
        m = jnp.max(dist, axis=1, keepdims=True)
        far_new = jnp.min(jnp.where(dist == m, iota, n), axis=1, keepdims=True)
        return dist, far_new

    dist0 = jnp.full((b, n), 1e10, dtype=_F32)
    far0 = jnp.zeros((b, 1), dtype=jnp.int32)
    jax.lax.fori_loop(0, s, body, (dist0, far0))


def _ballq_kernel(coord_ref, ce_ref, i1_ref, i2_ref, i3_ref):
    x = coord_ref[0]
    c = ce_ref[0]
    n = x.shape[1]
    s = c.shape[0]
    m = jax.lax.dot_general(c, x, (((1,), (0,)), ((), ())),
                            preferred_element_type=_F32)
    c2 = jnp.sum(c * c, axis=1, keepdims=True)
    x2 = jnp.sum(x * x, axis=0, keepdims=True)
    sqr = -2.0 * m + c2 + x2
    iota = jax.lax.broadcasted_iota(jnp.int32, (s, n), 1)
    for out_ref, r, k in ((i1_ref, _RS[0], _KS[0]),
                          (i2_ref, _RS[1], _KS[1]),
                          (i3_ref, _RS[2], _KS[2])):
        in_r = sqr <= r * r
        j0 = jnp.min(jnp.where(in_r, iota, n), axis=1, keepdims=True)
        iota_k = jax.lax.broadcasted_iota(jnp.int32, (s, k), 1)
        acc0 = jnp.where(iota_k == 0, j0, 0)

        def body(t, carry, j0=j0, iota_k=iota_k, in_r=in_r):
            j_prev, acc = carry
            cand = jnp.where(in_r & (iota > j_prev), iota, n)
            j = jnp.min(cand, axis=1, keepdims=True)
            acc = jnp.where(iota_k == t, jnp.where(j == n, j0, j), acc)
            return j, acc

        _, acc = jax.lax.fori_loop(1, k, body, (j0, acc0))
        out_ref[0] = acc


def _pre_kernel(ft_ref, ct_ref, wf1_ref, wc1_ref, wf2_ref, wc2_ref,
                wf3_ref, wc3_ref, p1_ref, p2_ref, p3_ref):
    f = ft_ref[0]
    c = ct_ref[0]
    for wf_ref, wc_ref, p_ref in ((wf1_ref, wc1_ref, p1_ref),
                                  (wf2_ref, wc2_ref, p2_ref),
                                  (wf3_ref, wc3_ref, p3_ref)):
        p_ref[0] = (jnp.dot(f, wf_ref[...], preferred_element_type=_F32)
                    + jnp.dot(c, wc_ref[...], preferred_element_type=_F32))


def _acc_stats(y, s1_ref, s2_ref):
    ps1 = jnp.sum(y, axis=0, keepdims=True)
    ps2 = jnp.sum(y * y, axis=0, keepdims=True)
    first = (pl.program_id(0) == 0) & (pl.program_id(1) == 0)

    @pl.when(first)
    def _():
        s1_ref[...] = ps1
        s2_ref[...] = ps2

    @pl.when(jnp.logical_not(first))
    def _():
        s1_ref[...] += ps1
        s2_ref[...] += ps2


def _g1_kernel(p_ref, idx_ref, ce_ref, wc_ref, bb_ref, y_ref, s1_ref, s2_ref):
    rows = idx_ref.shape[2]
    n = p_ref.shape[1]
    idxv = idx_ref[0, 0]
    iota = jax.lax.broadcasted_iota(jnp.int32, (rows, n), 1)
    oh = (iota == idxv).astype(_F32)
    g = jnp.dot(oh, p_ref[0], preferred_element_type=_F32)
    q = jnp.dot(ce_ref[0, 0], wc_ref[...], preferred_element_type=_F32)
    y = g - q + bb_ref[...]
    y_ref[0, 0] = y
    _acc_stats(y, s1_ref, s2_ref)


def _l2_kernel(y_ref, sc_ref, sh_ref, w_ref, bb_ref, o_ref, s1_ref, s2_ref):
    y = y_ref[0, 0]
    z = jnp.maximum(y * sc_ref[...] + sh_ref[...], 0.0)
    o = jnp.dot(z, w_ref[...], preferred_element_type=_F32) + bb_ref[...]
    o_ref[0, 0] = o
    _acc_stats(o, s1_ref, s2_ref)


def _l3max_kernel(y_ref, sc_ref, sh_ref, w_ref, bb_ref, mx_ref, s1_ref,
                  s2_ref, *, k):
    y = y_ref[0, 0]
    z = jnp.maximum(y * sc_ref[...] + sh_ref[...], 0.0)
    o = jnp.dot(z, w_ref[...], preferred_element_type=_F32) + bb_ref[...]
    _acc_stats(o, s1_ref, s2_ref)
    rows, d3 = o.shape
    mx_ref[0, 0] = jnp.max(o.reshape(rows // k, k, d3), axis=1)


def _epi_kernel(mx_ref, sc_ref, sh_ref, o_ref):
    v = mx_ref[0]
    o_ref[0] = jnp.maximum(v * sc_ref[...] + sh_ref[...], 0.0)


def _scale_shift(s1, s2, cnt, g, bt):
    mean = s1 / cnt
    var = s2 / cnt - mean * mean
    scale = (g[None, :] / jnp.sqrt(var + 1e-5))
    shift = bt[None, :] - mean * scale
    return scale, shift


def kernel(feature, coord, params):
    b, d_in, n = feature.shape
    s = _NG

    ce_t = pl.pallas_call(
        _fps_kernel,
        out_shape=jax.ShapeDtypeStruct((b, s, 3), _F32),
    )(coord)

    sq = min(256, s)
    ballq = pl.pallas_call(
        _ballq_kernel,
        grid=(b, s // sq),
        in_specs=[
            pl.BlockSpec((1, 3, n), lambda i, j: (i, 0, 0)),
            pl.BlockSpec((1, sq, 3), lambda i, j: (i, j, 0)),
        ],
        out_specs=[
            pl.BlockSpec((1, sq, _KS[0]), lambda i, j: (i, j, 0)),
            pl.BlockSpec((1, sq, _KS[1]), lambda i, j: (i, j, 0)),
            pl.BlockSpec((1, sq, _KS[2]), lambda i, j: (i, j, 0)),
        ],
        out_shape=[
            jax.ShapeDtypeStruct((b, s, _KS[0]), jnp.int32),
            jax.ShapeDtypeStruct((b, s, _KS[1]), jnp.int32),
            jax.ShapeDtypeStruct((b, s, _KS[2]), jnp.int32),
        ],
    )(coord, ce_t)

    d1s, d2s, d3s = [], [], []
    wf, wc, w2, w3 = [], [], [], []
    bias = []
    for blk in params:
        (w1_, b1_, g1_, t1_), (w2_, b2_, g2_, t2_), (w3_, b3_, g3_, t3_) = blk
        d1s.append(w1_.shape[0])
        d2s.append(w2_.shape[0])
        d3s.append(w3_.shape[0])
        wf.append(jnp.transpose(w1_[:, :d_in]))
        wc.append(jnp.transpose(w1_[:, d_in:]))
        w2.append(jnp.transpose(w2_))
        w3.append(jnp.transpose(w3_))
        bias.append(((b1_[None, :], g1_, t1_),
                     (b2_[None, :], g2_, t2_),
                     (b3_[None, :], g3_, t3_)))

    feat_t = jnp.transpose(feature, (0, 2, 1))
    coord_t = jnp.transpose(coord, (0, 2, 1))
    p_all = pl.pallas_call(
        _pre_kernel,
        grid=(b,),
        in_specs=[
            pl.BlockSpec((1, n, d_in), lambda i: (i, 0, 0)),
            pl.BlockSpec((1, n, 3), lambda i: (i, 0, 0)),
        ] + [pl.BlockSpec(w.shape, lambda i: (0, 0))
             for pair in zip(wf, wc) for w in pair],
        out_specs=[pl.BlockSpec((1, n, d1), lambda i: (i, 0, 0))
                   for d1 in d1s],
        out_shape=[jax.ShapeDtypeStruct((b, n, d1), _F32) for d1 in d1s],
    )(feat_t, coord_t, wf[0], wc[0], wf[1], wc[1], wf[2], wc[2])

    outs = []
    for br in range(3):
        k = _KS[br]
        d1, d2, d3 = d1s[br], d2s[br], d3s[br]
        sblk = _ROWS // k
        nsb = s // sblk
        cnt = float(b * s * k)
        idx = ballq[br]
        idx_col = idx.reshape(b, nsb, _ROWS, 1)
        ce_rep = jnp.broadcast_to(
            ce_t.reshape(b, nsb, sblk, 1, 3),
            (b, nsb, sblk, k, 3)).reshape(b, nsb, _ROWS, 3)

        y1, s1, s2 = pl.pallas_call(
            _g1_kernel,
            grid=(b, nsb),
            in_specs=[
                pl.BlockSpec((1, n, d1), lambda i, j: (i, 0, 0)),
                pl.BlockSpec((1, 1, _ROWS, 1), lambda i, j: (i, j, 0, 0)),
                pl.BlockSpec((1, 1, _ROWS, 3), lambda i, j: (i, j, 0, 0)),
                pl.BlockSpec((3, d1), lambda i, j: (0, 0)),
                pl.BlockSpec((1, d1), lambda i, j: (0, 0)),
            ],
            out_specs=[
                pl.BlockSpec((1, 1, _ROWS, d1), lambda i, j: (i, j, 0, 0)),
                pl.BlockSpec((1, d1), lambda i, j: (0, 0)),
                pl.BlockSpec((1, d1), lambda i, j: (0, 0)),
            ],
            out_shape=[
                jax.ShapeDtypeStruct((b, nsb, _ROWS, d1), _F32),
                jax.ShapeDtypeStruct((1, d1), _F32),
                jax.ShapeDtypeStruct((1, d1), _F32),
            ],
        )(p_all[br], idx_col, ce_rep, wc[br], bias[br][0][0])
        sc1, sh1 = _scale_shift(s1, s2, cnt, bias[br][0][1], bias[br][0][2])

        y2, s1, s2 = pl.pallas_call(
            _l2_kernel,
            grid=(b, nsb),
            in_specs=[
                pl.BlockSpec((1, 1, _ROWS, d1), lambda i, j: (i, j, 0, 0)),
                pl.BlockSpec((1, d1), lambda i, j: (0, 0)),
                pl.BlockSpec((1, d1), lambda i, j: (0, 0)),
                pl.BlockSpec((d1, d2), lambda i, j: (0, 0)),
                pl.BlockSpec((1, d2), lambda i, j: (0, 0)),
            ],
            out_specs=[
                pl.BlockSpec((1, 1, _ROWS, d2), lambda i, j: (i, j, 0, 0)),
                pl.BlockSpec((1, d2), lambda i, j: (0, 0)),
                pl.BlockSpec((1, d2), lambda i, j: (0, 0)),
            ],
            out_shape=[
                jax.ShapeDtypeStruct((b, nsb, _ROWS, d2), _F32),
                jax.ShapeDtypeStruct((1, d2), _F32),
                jax.ShapeDtypeStruct((1, d2), _F32),
            ],
        )(y1, sc1, sh1, w2[br], bias[br][1][0])
        sc2, sh2 = _scale_shift(s1, s2, cnt, bias[br][1][1], bias[br][1][2])

        mx, s1, s2 = pl.pallas_call(
            functools.partial(_l3max_kernel, k=k),
            grid=(b, nsb),
            in_specs=[
                pl.BlockSpec((1, 1, _ROWS, d2), lambda i, j: (i, j, 0, 0)),
                pl.BlockSpec((1, d2), lambda i, j: (0, 0)),
                pl.BlockSpec((1, d2), lambda i, j: (0, 0)),
                pl.BlockSpec((d2, d3), lambda i, j: (0, 0)),
                pl.BlockSpec((1, d3), lambda i, j: (0, 0)),
            ],
            out_specs=[
                pl.BlockSpec((1, 1, sblk, d3), lambda i, j: (i, j, 0, 0)),
                pl.BlockSpec((1, d3), lambda i, j: (0, 0)),
                pl.BlockSpec((1, d3), lambda i, j: (0, 0)),
            ],
            out_shape=[
                jax.ShapeDtypeStruct((b, nsb, sblk, d3), _F32),
                jax.ShapeDtypeStruct((1, d3), _F32),
                jax.ShapeDtypeStruct((1, d3), _F32),
            ],
        )(y2, sc2, sh2, w3[br], bias[br][2][0])
        sc3, sh3 = _scale_shift(s1, s2, cnt, bias[br][2][1], bias[br][2][2])

        z = pl.pallas_call(
            _epi_kernel,
            grid=(b,),
            in_specs=[
                pl.BlockSpec((1, nsb, sblk, d3), lambda i: (i, 0, 0, 0)),
                pl.BlockSpec((1, d3), lambda i: (0, 0)),
                pl.BlockSpec((1, d3), lambda i: (0, 0)),
            ],
            out_specs=pl.BlockSpec((1, nsb, sblk, d3), lambda i: (i, 0, 0, 0)),
            out_shape=jax.ShapeDtypeStruct((b, nsb, sblk, d3), _F32),
        )(mx, sc3, sh3)
        outs.append(jnp.transpose(z.reshape(b, s, d3), (0, 2, 1)))

    out1 = jnp.concatenate(outs, axis=1)
    out2 = jnp.transpose(ce_t, (0, 2, 1))
    return out1, out2


# speedup vs baseline: 6.8881x; 1.0220x over previous
import functools

import jax
import jax.numpy as jnp
from jax.experimental import pallas as pl
from jax.experimental.pallas import tpu as pltpu

_NG = 1024
_KS = (16, 32, 64)
_RS = (0.1, 0.2, 0.4)
_ROWS = 512

_F32 = jnp.float32


def _fps_kernel(coord_ref, ce_ref):
    b, _, n = coord_ref.shape
    s = ce_ref.shape[1]
    x0 = coord_ref[:, 0, :]
    x1 = coord_ref[:, 1, :]
    x2 = coord_ref[:, 2, :]
    iota = jax.lax.broadcasted_iota(jnp.int32, (b, n), 1)

    def body(i, carry):
        dist, far = carry
        oh = (iota == far).astype(_F32)
        c0 = jnp.sum(x0 * oh, axis=1, keepdims=True)
        c1 = jnp.sum(x1 * oh, axis=1, keepdims=True)
        c2 = jnp.sum(x2 * oh, axis=1, keepdims=True)
        cent = jnp.concatenate([c0, c1, c2], axis=1)
        ce_ref[:, pl.ds(i, 1), :] = cent[:, None, :]
        d0 = x0 - c0
        d1 = x1 - c1
        d2 = x2 - c2
        d = (d0 * d0 + d1 * d1) + d2 * d2
        dist = jnp.minimum(dist, d)
        m = jnp.max(dist, axis=1, keepdims=True)
        far_new = jnp.min(jnp.where(dist == m, iota, n), axis=1, keepdims=True)
        return dist, far_new

    dist0 = jnp.full((b, n), 1e10, dtype=_F32)
    far0 = jnp.zeros((b, 1), dtype=jnp.int32)
    jax.lax.fori_loop(0, s, body, (dist0, far0))


def _ballq_kernel(coord_ref, ce_ref, i1_ref, i2_ref, i3_ref):
    x = coord_ref[0]
    c = ce_ref[0]
    n = x.shape[1]
    s = c.shape[0]
    m = jax.lax.dot_general(c, x, (((1,), (0,)), ((), ())),
                            preferred_element_type=_F32)
    c2 = jnp.sum(c * c, axis=1, keepdims=True)
    x2 = jnp.sum(x * x, axis=0, keepdims=True)
    sqr = -2.0 * m + c2 + x2
    iota = jax.lax.broadcasted_iota(jnp.int32, (s, n), 1)
    for out_ref, r, k in ((i1_ref, _RS[0], _KS[0]),
                          (i2_ref, _RS[1], _KS[1]),
                          (i3_ref, _RS[2], _KS[2])):
        c0 = jnp.where(sqr <= r * r, iota, n)
        j0 = jnp.min(c0, axis=1, keepdims=True)
        iota_k = jax.lax.broadcasted_iota(jnp.int32, (s, k), 1)
        acc0 = jnp.where(iota_k == 0, j0, 0)

        def body(t, carry, j0=j0, iota_k=iota_k, c0=c0):
            j_prev, acc = carry
            j = jnp.min(jnp.where(c0 > j_prev, c0, n), axis=1, keepdims=True)
            acc = jnp.where(iota_k == t, jnp.where(j == n, j0, j), acc)
            return j, acc

        _, acc = jax.lax.fori_loop(1, k, body, (j0, acc0))
        out_ref[0] = acc


def _pre_kernel(ft_ref, ct_ref, wf1_ref, wc1_ref, wf2_ref, wc2_ref,
                wf3_ref, wc3_ref, p1_ref, p2_ref, p3_ref):
    f = ft_ref[0]
    c = ct_ref[0]
    for wf_ref, wc_ref, p_ref in ((wf1_ref, wc1_ref, p1_ref),
                                  (wf2_ref, wc2_ref, p2_ref),
                                  (wf3_ref, wc3_ref, p3_ref)):
        p_ref[0] = (jnp.dot(f, wf_ref[...], preferred_element_type=_F32)
                    + jnp.dot(c, wc_ref[...], preferred_element_type=_F32))


def _acc_stats(y, s1_ref, s2_ref):
    ps1 = jnp.sum(y, axis=0, keepdims=True)
    ps2 = jnp.sum(y * y, axis=0, keepdims=True)
    first = (pl.program_id(0) == 0) & (pl.program_id(1) == 0)

    @pl.when(first)
    def _():
        s1_ref[...] = ps1
        s2_ref[...] = ps2

    @pl.when(jnp.logical_not(first))
    def _():
        s1_ref[...] += ps1
        s2_ref[...] += ps2


def _g1_kernel(p_ref, idx_ref, ce_ref, wc_ref, bb_ref, y_ref, s1_ref, s2_ref):
    rows = idx_ref.shape[2]
    n = p_ref.shape[1]
    idxv = idx_ref[0, 0]
    iota = jax.lax.broadcasted_iota(jnp.int32, (rows, n), 1)
    oh = (iota == idxv).astype(_F32)
    g = jnp.dot(oh, p_ref[0], preferred_element_type=_F32)
    q = jnp.dot(ce_ref[0, 0], wc_ref[...], preferred_element_type=_F32)
    y = g - q + bb_ref[...]
    y_ref[0, 0] = y
    _acc_stats(y, s1_ref, s2_ref)


def _l2_kernel(y_ref, sc_ref, sh_ref, w_ref, bb_ref, o_ref, s1_ref, s2_ref):
    y = y_ref[0, 0]
    z = jnp.maximum(y * sc_ref[...] + sh_ref[...], 0.0)
    o = jnp.dot(z, w_ref[...], preferred_element_type=_F32) + bb_ref[...]
    o_ref[0, 0] = o
    _acc_stats(o, s1_ref, s2_ref)


def _l3max_kernel(y_ref, sc_ref, sh_ref, w_ref, bb_ref, mx_ref, s1_ref,
                  s2_ref, *, k):
    y = y_ref[0, 0]
    z = jnp.maximum(y * sc_ref[...] + sh_ref[...], 0.0)
    o = jnp.dot(z, w_ref[...], preferred_element_type=_F32) + bb_ref[...]
    _acc_stats(o, s1_ref, s2_ref)
    rows, d3 = o.shape
    mx_ref[0, 0] = jnp.max(o.reshape(rows // k, k, d3), axis=1)


def _epi_kernel(mx_ref, sc_ref, sh_ref, o_ref):
    v = mx_ref[0]
    o_ref[0] = jnp.maximum(v * sc_ref[...] + sh_ref[...], 0.0)


def _scale_shift(s1, s2, cnt, g, bt):
    mean = s1 / cnt
    var = s2 / cnt - mean * mean
    scale = (g[None, :] / jnp.sqrt(var + 1e-5))
    shift = bt[None, :] - mean * scale
    return scale, shift


def kernel(feature, coord, params):
    b, d_in, n = feature.shape
    s = _NG

    ce_t = pl.pallas_call(
        _fps_kernel,
        out_shape=jax.ShapeDtypeStruct((b, s, 3), _F32),
    )(coord)

    sq = min(256, s)
    ballq = pl.pallas_call(
        _ballq_kernel,
        grid=(b, s // sq),
        compiler_params=pltpu.CompilerParams(
            dimension_semantics=("parallel", "parallel")),
        in_specs=[
            pl.BlockSpec((1, 3, n), lambda i, j: (i, 0, 0)),
            pl.BlockSpec((1, sq, 3), lambda i, j: (i, j, 0)),
        ],
        out_specs=[
            pl.BlockSpec((1, sq, _KS[0]), lambda i, j: (i, j, 0)),
            pl.BlockSpec((1, sq, _KS[1]), lambda i, j: (i, j, 0)),
            pl.BlockSpec((1, sq, _KS[2]), lambda i, j: (i, j, 0)),
        ],
        out_shape=[
            jax.ShapeDtypeStruct((b, s, _KS[0]), jnp.int32),
            jax.ShapeDtypeStruct((b, s, _KS[1]), jnp.int32),
            jax.ShapeDtypeStruct((b, s, _KS[2]), jnp.int32),
        ],
    )(coord, ce_t)

    d1s, d2s, d3s = [], [], []
    wf, wc, w2, w3 = [], [], [], []
    bias = []
    for blk in params:
        (w1_, b1_, g1_, t1_), (w2_, b2_, g2_, t2_), (w3_, b3_, g3_, t3_) = blk
        d1s.append(w1_.shape[0])
        d2s.append(w2_.shape[0])
        d3s.append(w3_.shape[0])
        wf.append(jnp.transpose(w1_[:, :d_in]))
        wc.append(jnp.transpose(w1_[:, d_in:]))
        w2.append(jnp.transpose(w2_))
        w3.append(jnp.transpose(w3_))
        bias.append(((b1_[None, :], g1_, t1_),
                     (b2_[None, :], g2_, t2_),
                     (b3_[None, :], g3_, t3_)))

    feat_t = jnp.transpose(feature, (0, 2, 1))
    coord_t = jnp.transpose(coord, (0, 2, 1))
    p_all = pl.pallas_call(
        _pre_kernel,
        grid=(b,),
        in_specs=[
            pl.BlockSpec((1, n, d_in), lambda i: (i, 0, 0)),
            pl.BlockSpec((1, n, 3), lambda i: (i, 0, 0)),
        ] + [pl.BlockSpec(w.shape, lambda i: (0, 0))
             for pair in zip(wf, wc) for w in pair],
        out_specs=[pl.BlockSpec((1, n, d1), lambda i: (i, 0, 0))
                   for d1 in d1s],
        out_shape=[jax.ShapeDtypeStruct((b, n, d1), _F32) for d1 in d1s],
    )(feat_t, coord_t, wf[0], wc[0], wf[1], wc[1], wf[2], wc[2])

    outs = []
    for br in range(3):
        k = _KS[br]
        d1, d2, d3 = d1s[br], d2s[br], d3s[br]
        sblk = _ROWS // k
        nsb = s // sblk
        cnt = float(b * s * k)
        idx = ballq[br]
        idx_col = idx.reshape(b, nsb, _ROWS, 1)
        ce_rep = jnp.broadcast_to(
            ce_t.reshape(b, nsb, sblk, 1, 3),
            (b, nsb, sblk, k, 3)).reshape(b, nsb, _ROWS, 3)

        y1, s1, s2 = pl.pallas_call(
            _g1_kernel,
            grid=(b, nsb),
            in_specs=[
                pl.BlockSpec((1, n, d1), lambda i, j: (i, 0, 0)),
                pl.BlockSpec((1, 1, _ROWS, 1), lambda i, j: (i, j, 0, 0)),
                pl.BlockSpec((1, 1, _ROWS, 3), lambda i, j: (i, j, 0, 0)),
                pl.BlockSpec((3, d1), lambda i, j: (0, 0)),
                pl.BlockSpec((1, d1), lambda i, j: (0, 0)),
            ],
            out_specs=[
                pl.BlockSpec((1, 1, _ROWS, d1), lambda i, j: (i, j, 0, 0)),
                pl.BlockSpec((1, d1), lambda i, j: (0, 0)),
                pl.BlockSpec((1, d1), lambda i, j: (0, 0)),
            ],
            out_shape=[
                jax.ShapeDtypeStruct((b, nsb, _ROWS, d1), _F32),
                jax.ShapeDtypeStruct((1, d1), _F32),
                jax.ShapeDtypeStruct((1, d1), _F32),
            ],
        )(p_all[br], idx_col, ce_rep, wc[br], bias[br][0][0])
        sc1, sh1 = _scale_shift(s1, s2, cnt, bias[br][0][1], bias[br][0][2])

        y2, s1, s2 = pl.pallas_call(
            _l2_kernel,
            grid=(b, nsb),
            in_specs=[
                pl.BlockSpec((1, 1, _ROWS, d1), lambda i, j: (i, j, 0, 0)),
                pl.BlockSpec((1, d1), lambda i, j: (0, 0)),
                pl.BlockSpec((1, d1), lambda i, j: (0, 0)),
                pl.BlockSpec((d1, d2), lambda i, j: (0, 0)),
                pl.BlockSpec((1, d2), lambda i, j: (0, 0)),
            ],
            out_specs=[
                pl.BlockSpec((1, 1, _ROWS, d2), lambda i, j: (i, j, 0, 0)),
                pl.BlockSpec((1, d2), lambda i, j: (0, 0)),
                pl.BlockSpec((1, d2), lambda i, j: (0, 0)),
            ],
            out_shape=[
                jax.ShapeDtypeStruct((b, nsb, _ROWS, d2), _F32),
                jax.ShapeDtypeStruct((1, d2), _F32),
                jax.ShapeDtypeStruct((1, d2), _F32),
            ],
        )(y1, sc1, sh1, w2[br], bias[br][1][0])
        sc2, sh2 = _scale_shift(s1, s2, cnt, bias[br][1][1], bias[br][1][2])

        mx, s1, s2 = pl.pallas_call(
            functools.partial(_l3max_kernel, k=k),
            grid=(b, nsb),
            in_specs=[
                pl.BlockSpec((1, 1, _ROWS, d2), lambda i, j: (i, j, 0, 0)),
                pl.BlockSpec((1, d2), lambda i, j: (0, 0)),
                pl.BlockSpec((1, d2), lambda i, j: (0, 0)),
                pl.BlockSpec((d2, d3), lambda i, j: (0, 0)),
                pl.BlockSpec((1, d3), lambda i, j: (0, 0)),
            ],
            out_specs=[
                pl.BlockSpec((1, 1, sblk, d3), lambda i, j: (i, j, 0, 0)),
                pl.BlockSpec((1, d3), lambda i, j: (0, 0)),
                pl.BlockSpec((1, d3), lambda i, j: (0, 0)),
            ],
            out_shape=[
                jax.ShapeDtypeStruct((b, nsb, sblk, d3), _F32),
                jax.ShapeDtypeStruct((1, d3), _F32),
                jax.ShapeDtypeStruct((1, d3), _F32),
            ],
        )(y2, sc2, sh2, w3[br], bias[br][2][0])
        sc3, sh3 = _scale_shift(s1, s2, cnt, bias[br][2][1], bias[br][2][2])

        z = pl.pallas_call(
            _epi_kernel,
            grid=(b,),
            in_specs=[
                pl.BlockSpec((1, nsb, sblk, d3), lambda i: (i, 0, 0, 0)),
                pl.BlockSpec((1, d3), lambda i: (0, 0)),
                pl.BlockSpec((1, d3), lambda i: (0, 0)),
            ],
            out_specs=pl.BlockSpec((1, nsb, sblk, d3), lambda i: (i, 0, 0, 0)),
            out_shape=jax.ShapeDtypeStruct((b, nsb, sblk, d3), _F32),
        )(mx, sc3, sh3)
        outs.append(jnp.transpose(z.reshape(b, s, d3), (0, 2, 1)))

    out1 = jnp.concatenate(outs, axis=1)
    out2 = jnp.transpose(ce_t, (0, 2, 1))
    return out1, out2
